# baseline (speedup 1.0000x reference)
import functools

import jax
import jax.numpy as jnp
from jax import lax
from jax.experimental import pallas as pl
from jax.experimental.pallas import tpu as pltpu
from jax.experimental.pallas import tpu_sc as plsc

_IDX_CHUNK = 128


@functools.lru_cache(maxsize=None)
def _make_sc_kernel(B, D):
    info = plsc.get_sparse_core_info()
    NC, NS, L = info.num_cores, info.num_subcores, info.num_lanes
    NW = NC * NS
    b_per_w = B // NW
    n_chunks = b_per_w // _IDX_CHUNK
    n_groups = b_per_w // L

    mesh = plsc.VectorSubcoreMesh(core_axis_name="c", subcore_axis_name="s")

    @functools.partial(
        pl.kernel,
        mesh=mesh,
        out_type=jax.ShapeDtypeStruct((B,), jnp.float32),
        compiler_params=pltpu.CompilerParams(
            needs_layout_passes=False, use_tc_tiling_on_sc=False),
        scratch_types=[
            pltpu.VMEM((n_chunks, _IDX_CHUNK), jnp.int32),
            pltpu.VMEM((n_chunks, _IDX_CHUNK), jnp.int32),
            pltpu.VMEM((b_per_w, D), jnp.float32),
            pltpu.VMEM((b_per_w, D), jnp.float32),
            pltpu.VMEM((b_per_w,), jnp.float32),
            pltpu.SemaphoreType.DMA,
        ],
    )
    def sc_kernel(uidx_hbm, iidx_hbm, utab_hbm, itab_hbm, out_hbm,
                  uidx_v, iidx_v, urows, irows, out_v, sem):
        wid = lax.axis_index("s") * NC + lax.axis_index("c")
        base = wid * b_per_w

        pltpu.sync_copy(uidx_hbm.at[pl.ds(wid * n_chunks, n_chunks)], uidx_v)
        pltpu.sync_copy(iidx_hbm.at[pl.ds(wid * n_chunks, n_chunks)], iidx_v)

        copies = []
        for j in range(n_chunks):
            dst = pl.ds(j * _IDX_CHUNK, _IDX_CHUNK)
            copies.append(
                pltpu.async_copy(utab_hbm.at[uidx_v.at[j]], urows.at[dst], sem))
            copies.append(
                pltpu.async_copy(itab_hbm.at[iidx_v.at[j]], irows.at[dst], sem))
        for c in copies:
            c.wait()

        def group_body(g, carry):
            rows = g * L + lax.iota(jnp.int32, L)
            acc = jnp.zeros((L,), jnp.float32)
            for d in range(D):
                cols = jnp.full((L,), d, jnp.int32)
                u = plsc.load_gather(urows, [rows, cols])
                v = plsc.load_gather(irows, [rows, cols])
                acc = acc + u * v
            out_v[pl.ds(g * L, L)] = acc
            return carry

        lax.fori_loop(0, n_groups, group_body, 0)

        pltpu.sync_copy(out_v, out_hbm.at[pl.ds(base, b_per_w)])

    return sc_kernel


def kernel(user_idx, item_idx, user_table, item_table):
    B = user_idx.shape[0]
    D = user_table.shape[1]
    uidx = user_idx.astype(jnp.int32).reshape(-1, _IDX_CHUNK)
    iidx = item_idx.astype(jnp.int32).reshape(-1, _IDX_CHUNK)
    out = _make_sc_kernel(B, D)(uidx, iidx, user_table, item_table)
    return out.reshape(B, 1)


# speedup vs baseline: 1.5642x; 1.5642x over previous
import functools

import jax
import jax.numpy as jnp
from jax import lax
from jax.experimental import pallas as pl
from jax.experimental.pallas import tpu as pltpu
from jax.experimental.pallas import tpu_sc as plsc

_NBUF = 2


@functools.lru_cache(maxsize=None)
def _make_sc_kernel(B, D):
    info = plsc.get_sparse_core_info()
    NC, NS, L = info.num_cores, info.num_subcores, info.num_lanes
    NW = NC * NS
    b_per_w = B // NW
    n_groups = b_per_w // L

    mesh = plsc.VectorSubcoreMesh(core_axis_name="c", subcore_axis_name="s")

    @functools.partial(
        pl.kernel,
        mesh=mesh,
        out_type=jax.ShapeDtypeStruct((B,), jnp.float32),
        compiler_params=pltpu.CompilerParams(needs_layout_passes=False),
        scratch_types=[
            pltpu.VMEM((b_per_w,), jnp.int32),
            pltpu.VMEM((b_per_w,), jnp.int32),
            pltpu.VMEM((_NBUF * L, D), jnp.float32),
            pltpu.VMEM((_NBUF * L, D), jnp.float32),
            pltpu.VMEM((b_per_w,), jnp.float32),
            pltpu.SemaphoreType.DMA,
        ],
    )
    def sc_kernel(uidx_hbm, iidx_hbm, utab_hbm, itab_hbm, out_hbm,
                  uidx_v, iidx_v, urows, irows, out_v, sem):
        wid = lax.axis_index("s") * NC + lax.axis_index("c")
        base = wid * b_per_w

        pltpu.sync_copy(uidx_hbm.at[pl.ds(base, b_per_w)], uidx_v)
        pltpu.sync_copy(iidx_hbm.at[pl.ds(base, b_per_w)], iidx_v)

        def enqueue_group(g):
            slot = lax.rem(g, _NBUF) * L
            iv_u = uidx_v[pl.ds(g * L, L)]
            iv_i = iidx_v[pl.ds(g * L, L)]
            for l in range(L):
                pltpu.async_copy(
                    utab_hbm.at[iv_u[l]], urows.at[slot + l], sem)
                pltpu.async_copy(
                    itab_hbm.at[iv_i[l]], irows.at[slot + l], sem)

        def drain_group():
            pltpu.make_async_copy(
                utab_hbm.at[pl.ds(0, L)], urows.at[pl.ds(0, L)], sem).wait()
            pltpu.make_async_copy(
                itab_hbm.at[pl.ds(0, L)], irows.at[pl.ds(0, L)], sem).wait()

        enqueue_group(0)

        def group_body(g, carry):
            @pl.when(g + 1 < n_groups)
            def _():
                enqueue_group(g + 1)

            drain_group()

            slot = lax.rem(g, _NBUF) * L
            rows = slot + lax.iota(jnp.int32, L)
            acc = jnp.zeros((L,), jnp.float32)
            for d in range(D):
                cols = jnp.full((L,), d, jnp.int32)
                u = plsc.load_gather(urows, [rows, cols])
                v = plsc.load_gather(irows, [rows, cols])
                acc = acc + u * v
            out_v[pl.ds(g * L, L)] = acc
            return carry

        lax.fori_loop(0, n_groups, group_body, 0)

        pltpu.sync_copy(out_v, out_hbm.at[pl.ds(base, b_per_w)])

    return sc_kernel


def kernel(user_idx, item_idx, user_table, item_table):
    B = user_idx.shape[0]
    D = user_table.shape[1]
    uidx = user_idx.astype(jnp.int32)
    iidx = item_idx.astype(jnp.int32)
    out = _make_sc_kernel(B, D)(uidx, iidx, user_table, item_table)
    return out.reshape(B, 1)
